# baseline (speedup 1.0000x reference)
import math

import jax
import jax.numpy as jnp
from jax.experimental import pallas as pl
from jax.experimental.pallas import tpu as pltpu
from jax.experimental.pallas import tpu_sc as plsc

_TOPK = 2
_CAP_FACTOR = 2.0


def _router_kernel(flat_ref, wg_ref, p_ref, g_ref, tgt_ref, fseg_ref,
                   gates_scr, pos_scr):
    e = pl.program_id(0)
    E, N = gates_scr.shape
    cap = p_ref.shape[1]
    dump = E * cap
    nseg = fseg_ref.shape[0]
    segw = fseg_ref.shape[2]

    @pl.when(e == 0)
    def _compute_routing():
        logits = jax.lax.dot_general(
            wg_ref[...], flat_ref[...],
            dimension_numbers=(((1,), (1,)), ((), ())),
            preferred_element_type=jnp.float32)
        m = jnp.max(logits, axis=0, keepdims=True)
        ex = jnp.exp(logits - m)
        g = ex / jnp.sum(ex, axis=0, keepdims=True)
        ioe = jax.lax.broadcasted_iota(jnp.int32, (E, N), 0)
        v1 = jnp.max(g, axis=0, keepdims=True)
        i1 = jnp.min(jnp.where(g == v1, ioe, E), axis=0, keepdims=True)
        m1 = ioe == i1
        g2 = jnp.where(m1, -jnp.inf, g)
        v2 = jnp.max(g2, axis=0, keepdims=True)
        i2 = jnp.min(jnp.where(g2 == v2, ioe, E), axis=0, keepdims=True)
        m2 = ioe == i2
        mask = (m1 | m2).astype(jnp.int32)
        cums = mask
        sh = 1
        while sh < N:
            shifted = jnp.concatenate(
                [jnp.zeros((E, sh), jnp.int32), cums[:, :N - sh]], axis=1)
            cums = cums + shifted
            sh *= 2
        pos = cums * mask - 1
        pos_scr[...] = pos
        gates_scr[...] = g
        g_ref[...] = g
        pos1 = jnp.sum(jnp.where(m1, pos, 0), axis=0, keepdims=True)
        pos2 = jnp.sum(jnp.where(m2, pos, 0), axis=0, keepdims=True)
        tgt1 = jnp.where(pos1 < cap, i1 * cap + pos1, dump)
        tgt2 = jnp.where(pos2 < cap, i2 * cap + pos2, dump)
        ior = jax.lax.broadcasted_iota(jnp.int32, (nseg, N), 0)
        tgt_ref[...] = jnp.concatenate(
            [tgt1 * nseg + ior, tgt2 * nseg + ior], axis=0)
        for r in range(nseg):
            fseg_ref[r] = flat_ref[:, pl.ds(r * segw, segw)]

    pos_row = pos_scr[pl.ds(e, 1), :]
    ioc = jax.lax.broadcasted_iota(jnp.int32, (cap, N), 0)
    p_ref[0] = (ioc == pos_row).astype(jnp.bfloat16)


def _expert_kernel(p_ref, g_ref, gath_ref, w1_ref, b1_ref, w2_ref, b2_ref,
                   out_ref, acc_scr, gv_scr):
    e = pl.program_id(0)
    k = pl.program_id(1)
    K = pl.num_programs(1)
    HC = w1_ref.shape[1]

    @pl.when((e == 0) & (k == 0))
    def _zero_out():
        out_ref[...] = jnp.zeros_like(out_ref)

    @pl.when(k == 0)
    def _prep():
        acc_scr[...] = jnp.zeros_like(acc_scr)
        pf = p_ref[0].astype(jnp.float32)
        gv_scr[...] = jnp.sum(
            pf * g_ref[pl.ds(e, 1), :], axis=1, keepdims=True)

    hpre = jax.lax.dot_general(
        gath_ref[...].astype(jnp.bfloat16), w1_ref[0].astype(jnp.bfloat16),
        dimension_numbers=(((1,), (1,)), ((), ())),
        preferred_element_type=jnp.float32)
    hpre = hpre + b1_ref[pl.ds(e, 1), pl.ds(k * HC, HC)]
    h = 0.5 * hpre * (1.0 + jax.lax.erf(hpre * (1.0 / math.sqrt(2.0))))
    acc_scr[...] += jax.lax.dot_general(
        h.astype(jnp.bfloat16), w2_ref[0].astype(jnp.bfloat16),
        dimension_numbers=(((1,), (1,)), ((), ())),
        preferred_element_type=jnp.float32)

    @pl.when(k == K - 1)
    def _combine():
        gv = gv_scr[...]
        outc = jnp.where(gv > 0.0,
                         (acc_scr[...] + b2_ref[pl.ds(e, 1), :]) * gv,
                         0.0)
        out_ref[...] += jax.lax.dot_general(
            p_ref[0], outc.astype(jnp.bfloat16),
            dimension_numbers=(((0,), (0,)), ((), ())),
            preferred_element_type=jnp.float32)


def _sc_dispatch(fseg, tgt, n_slots):
    nseg, N, segw = fseg.shape
    nsub = nseg * N
    fsub = fseg.reshape(nsub, segw)
    W = 128
    nblk = N // W

    @pl.kernel(
        out_type=jax.ShapeDtypeStruct((n_slots * nseg, segw), jnp.float32),
        mesh=plsc.VectorSubcoreMesh(
            core_axis_name="core", subcore_axis_name="subcore"),
    )
    def sck(fsub_hbm, tgt_hbm, gath_hbm):
        for j in (0, 1):
            def body(x_vmem, i_vmem):
                pltpu.sync_copy(x_vmem, gath_hbm.at[i_vmem.at[0]])

            pltpu.emit_pipeline(
                body,
                grid=(nsub // W,),
                in_specs=[
                    pl.BlockSpec((W, segw), index_map=lambda i: (i, 0)),
                    pl.BlockSpec(
                        (1, W),
                        index_map=lambda i, j=j: (j * nseg + i // nblk,
                                                  i % nblk)),
                ],
                out_specs=[],
                core_axis_name="subcore",
                dimension_semantics=(pltpu.PARALLEL,),
            )(fsub_hbm, tgt_hbm)

    return sck(fsub, tgt)


def kernel(hidden_states, Wg, W1, b1, W2, b2):
    Bv, Tv, D = hidden_states.shape
    N = Bv * Tv
    E, H, _ = W1.shape
    cap = math.ceil(_CAP_FACTOR * N / E)
    HC = 1024
    K = H // HC

    flat = hidden_states.reshape(N, D)

    NSEG = 8
    SEGW = D // NSEG
    p, g, tgt, fseg = pl.pallas_call(
        _router_kernel,
        grid=(E,),
        in_specs=[
            pl.BlockSpec((N, D), lambda e: (0, 0)),
            pl.BlockSpec((E, D), lambda e: (0, 0)),
        ],
        out_specs=[
            pl.BlockSpec((1, cap, N), lambda e: (e, 0, 0)),
            pl.BlockSpec((E, N), lambda e: (0, 0)),
            pl.BlockSpec((2 * NSEG, N), lambda e: (0, 0)),
            pl.BlockSpec((NSEG, N, SEGW), lambda e: (0, 0, 0)),
        ],
        out_shape=[
            jax.ShapeDtypeStruct((E, cap, N), jnp.bfloat16),
            jax.ShapeDtypeStruct((E, N), jnp.float32),
            jax.ShapeDtypeStruct((2 * NSEG, N), jnp.int32),
            jax.ShapeDtypeStruct((NSEG, N, SEGW), jnp.float32),
        ],
        scratch_shapes=[
            pltpu.VMEM((E, N), jnp.float32),
            pltpu.VMEM((E, N), jnp.int32),
        ],
    )(flat, Wg)

    gath = _sc_dispatch(fseg, tgt, (E + 1) * cap).reshape((E + 1) * cap, D)

    final = pl.pallas_call(
        _expert_kernel,
        grid=(E, K),
        in_specs=[
            pl.BlockSpec((1, cap, N), lambda e, k: (e, 0, 0)),
            pl.BlockSpec((E, N), lambda e, k: (0, 0)),
            pl.BlockSpec((cap, D), lambda e, k: (e, 0)),
            pl.BlockSpec((1, HC, D), lambda e, k: (e, k, 0)),
            pl.BlockSpec((E, H), lambda e, k: (0, 0)),
            pl.BlockSpec((1, D, HC), lambda e, k: (e, 0, k)),
            pl.BlockSpec((E, D), lambda e, k: (0, 0)),
        ],
        out_specs=pl.BlockSpec((N, D), lambda e, k: (0, 0)),
        out_shape=jax.ShapeDtypeStruct((N, D), jnp.float32),
        scratch_shapes=[
            pltpu.VMEM((cap, D), jnp.float32),
            pltpu.VMEM((cap, 1), jnp.float32),
        ],
    )(p, g, gath, W1, b1, W2, b2)

    aux_loss = jnp.asarray(0.0, dtype=jnp.float32)
    return (final.reshape(Bv, Tv, D), aux_loss)


# speedup vs baseline: 1.3257x; 1.3257x over previous
import math

import jax
import jax.numpy as jnp
from jax.experimental import pallas as pl
from jax.experimental.pallas import tpu as pltpu

_TOPK = 2
_CAP_FACTOR = 2.0


def _router_kernel(flat_ref, wg_ref, p_ref, g_ref, gates_scr, pos_scr):
    e = pl.program_id(0)
    E, N = gates_scr.shape
    cap = p_ref.shape[1]

    @pl.when(e == 0)
    def _compute_routing():
        logits = jax.lax.dot_general(
            wg_ref[...], flat_ref[...],
            dimension_numbers=(((1,), (1,)), ((), ())),
            preferred_element_type=jnp.float32)
        m = jnp.max(logits, axis=0, keepdims=True)
        ex = jnp.exp(logits - m)
        g = ex / jnp.sum(ex, axis=0, keepdims=True)
        ioe = jax.lax.broadcasted_iota(jnp.int32, (E, N), 0)
        v1 = jnp.max(g, axis=0, keepdims=True)
        i1 = jnp.min(jnp.where(g == v1, ioe, E), axis=0, keepdims=True)
        m1 = ioe == i1
        g2 = jnp.where(m1, -jnp.inf, g)
        v2 = jnp.max(g2, axis=0, keepdims=True)
        i2 = jnp.min(jnp.where(g2 == v2, ioe, E), axis=0, keepdims=True)
        mask = (m1 | (ioe == i2)).astype(jnp.int32)
        cums = mask
        sh = 1
        while sh < N:
            shifted = jnp.concatenate(
                [jnp.zeros((E, sh), jnp.int32), cums[:, :N - sh]], axis=1)
            cums = cums + shifted
            sh *= 2
        pos_scr[...] = cums * mask - 1
        gates_scr[...] = g
        g_ref[...] = g

    pos_row = pos_scr[pl.ds(e, 1), :]
    ioc = jax.lax.broadcasted_iota(jnp.int32, (cap, N), 0)
    p_ref[0] = (ioc == pos_row).astype(jnp.bfloat16)


def _expert_kernel(flat_ref, p_ref, g_ref, w1_ref, b1_ref, w2_ref, b2_ref,
                   out_ref, fbf_scr, gath_scr, acc_scr, gv_scr):
    e = pl.program_id(0)
    k = pl.program_id(1)
    K = pl.num_programs(1)
    HC = w1_ref.shape[1]

    @pl.when((e == 0) & (k == 0))
    def _zero_out():
        out_ref[...] = jnp.zeros_like(out_ref)
        fbf_scr[...] = flat_ref[...].astype(jnp.bfloat16)

    @pl.when(k == 0)
    def _dispatch():
        gath_scr[...] = jax.lax.dot_general(
            p_ref[0], fbf_scr[...],
            dimension_numbers=(((1,), (0,)), ((), ())),
            preferred_element_type=jnp.float32).astype(jnp.bfloat16)
        acc_scr[...] = jnp.zeros_like(acc_scr)
        pf = p_ref[0].astype(jnp.float32)
        gv_scr[...] = jnp.sum(
            pf * g_ref[pl.ds(e, 1), :], axis=1, keepdims=True)

    hpre = jax.lax.dot_general(
        gath_scr[...], w1_ref[0].astype(jnp.bfloat16),
        dimension_numbers=(((1,), (1,)), ((), ())),
        preferred_element_type=jnp.float32)
    hpre = hpre + b1_ref[pl.ds(e, 1), pl.ds(k * HC, HC)]
    h = 0.5 * hpre * (1.0 + jax.lax.erf(hpre * (1.0 / math.sqrt(2.0))))
    acc_scr[...] += jax.lax.dot_general(
        h.astype(jnp.bfloat16), w2_ref[0].astype(jnp.bfloat16),
        dimension_numbers=(((1,), (1,)), ((), ())),
        preferred_element_type=jnp.float32)

    @pl.when(k == K - 1)
    def _combine():
        outc = (acc_scr[...] + b2_ref[pl.ds(e, 1), :]) * gv_scr[...]
        out_ref[...] += jax.lax.dot_general(
            p_ref[0], outc.astype(jnp.bfloat16),
            dimension_numbers=(((0,), (0,)), ((), ())),
            preferred_element_type=jnp.float32)


def kernel(hidden_states, Wg, W1, b1, W2, b2):
    Bv, Tv, D = hidden_states.shape
    N = Bv * Tv
    E, H, _ = W1.shape
    cap = math.ceil(_CAP_FACTOR * N / E)
    HC = 1024
    K = H // HC

    flat = hidden_states.reshape(N, D)

    p, g = pl.pallas_call(
        _router_kernel,
        grid=(E,),
        in_specs=[
            pl.BlockSpec((N, D), lambda e: (0, 0)),
            pl.BlockSpec((E, D), lambda e: (0, 0)),
        ],
        out_specs=[
            pl.BlockSpec((1, cap, N), lambda e: (e, 0, 0)),
            pl.BlockSpec((E, N), lambda e: (0, 0)),
        ],
        out_shape=[
            jax.ShapeDtypeStruct((E, cap, N), jnp.bfloat16),
            jax.ShapeDtypeStruct((E, N), jnp.float32),
        ],
        scratch_shapes=[
            pltpu.VMEM((E, N), jnp.float32),
            pltpu.VMEM((E, N), jnp.int32),
        ],
    )(flat, Wg)

    final = pl.pallas_call(
        _expert_kernel,
        grid=(E, K),
        in_specs=[
            pl.BlockSpec((N, D), lambda e, k: (0, 0)),
            pl.BlockSpec((1, cap, N), lambda e, k: (e, 0, 0)),
            pl.BlockSpec((E, N), lambda e, k: (0, 0)),
            pl.BlockSpec((1, HC, D), lambda e, k: (e, k, 0)),
            pl.BlockSpec((E, H), lambda e, k: (0, 0)),
            pl.BlockSpec((1, D, HC), lambda e, k: (e, 0, k)),
            pl.BlockSpec((E, D), lambda e, k: (0, 0)),
        ],
        out_specs=pl.BlockSpec((N, D), lambda e, k: (0, 0)),
        out_shape=jax.ShapeDtypeStruct((N, D), jnp.float32),
        scratch_shapes=[
            pltpu.VMEM((N, D), jnp.bfloat16),
            pltpu.VMEM((cap, D), jnp.bfloat16),
            pltpu.VMEM((cap, D), jnp.float32),
            pltpu.VMEM((cap, 1), jnp.float32),
        ],
    )(flat, p, g, W1, b1, W2, b2)

    aux_loss = jnp.asarray(0.0, dtype=jnp.float32)
    return (final.reshape(Bv, Tv, D), aux_loss)


# speedup vs baseline: 1.3948x; 1.0521x over previous
import math

import jax
import jax.numpy as jnp
from jax.experimental import pallas as pl
from jax.experimental.pallas import tpu as pltpu

_TOPK = 2
_CAP_FACTOR = 2.0


def _router_kernel(flat_ref, wg_ref, g_ref, pos_ref):
    E, N = g_ref.shape
    logits = jax.lax.dot_general(
        wg_ref[...], flat_ref[...],
        dimension_numbers=(((1,), (1,)), ((), ())),
        preferred_element_type=jnp.float32)
    m = jnp.max(logits, axis=0, keepdims=True)
    ex = jnp.exp(logits - m)
    g = ex / jnp.sum(ex, axis=0, keepdims=True)
    ioe = jax.lax.broadcasted_iota(jnp.int32, (E, N), 0)
    v1 = jnp.max(g, axis=0, keepdims=True)
    i1 = jnp.min(jnp.where(g == v1, ioe, E), axis=0, keepdims=True)
    m1 = ioe == i1
    g2 = jnp.where(m1, -jnp.inf, g)
    v2 = jnp.max(g2, axis=0, keepdims=True)
    i2 = jnp.min(jnp.where(g2 == v2, ioe, E), axis=0, keepdims=True)
    mask = (m1 | (ioe == i2)).astype(jnp.int32)
    cums = mask
    sh = 1
    while sh < N:
        shifted = jnp.concatenate(
            [jnp.zeros((E, sh), jnp.int32), cums[:, :N - sh]], axis=1)
        cums = cums + shifted
        sh *= 2
    pos_ref[...] = cums * mask - 1
    g_ref[...] = g


def _expert_kernel(flat_ref, g_ref, pos_ref, w1_ref, b1_ref, w2_ref, b2_ref,
                   out_ref, fbf_scr, p_scr, gath_scr, acc_scr, gv_scr):
    e = pl.program_id(0)
    k = pl.program_id(1)
    K = pl.num_programs(1)
    HC = w1_ref.shape[1]
    cap = p_scr.shape[0]
    N = p_scr.shape[1]

    @pl.when((e == 0) & (k == 0))
    def _zero_out():
        out_ref[...] = jnp.zeros_like(out_ref)
        fbf_scr[...] = flat_ref[...].astype(jnp.bfloat16)

    @pl.when(k == 0)
    def _dispatch():
        pos_row = pos_ref[pl.ds(e, 1), :]
        ioc = jax.lax.broadcasted_iota(jnp.int32, (cap, N), 0)
        pf = (ioc == pos_row).astype(jnp.float32)
        p_scr[...] = pf.astype(jnp.bfloat16)
        gv_scr[...] = jnp.sum(
            pf * g_ref[pl.ds(e, 1), :], axis=1, keepdims=True)
        gath_scr[...] = jax.lax.dot_general(
            p_scr[...], fbf_scr[...],
            dimension_numbers=(((1,), (0,)), ((), ())),
            preferred_element_type=jnp.float32).astype(jnp.bfloat16)
        acc_scr[...] = jnp.zeros_like(acc_scr)

    hpre = jax.lax.dot_general(
        gath_scr[...], w1_ref[0].astype(jnp.bfloat16),
        dimension_numbers=(((1,), (1,)), ((), ())),
        preferred_element_type=jnp.float32)
    hpre = hpre + b1_ref[pl.ds(e, 1), pl.ds(k * HC, HC)]
    h = 0.5 * hpre * (1.0 + jax.lax.erf(hpre * (1.0 / math.sqrt(2.0))))
    acc_scr[...] += jax.lax.dot_general(
        h.astype(jnp.bfloat16), w2_ref[0].astype(jnp.bfloat16),
        dimension_numbers=(((1,), (1,)), ((), ())),
        preferred_element_type=jnp.float32)

    @pl.when(k == K - 1)
    def _combine():
        outc = (acc_scr[...] + b2_ref[pl.ds(e, 1), :]) * gv_scr[...]
        out_ref[...] += jax.lax.dot_general(
            p_scr[...], outc.astype(jnp.bfloat16),
            dimension_numbers=(((0,), (0,)), ((), ())),
            preferred_element_type=jnp.float32)


def kernel(hidden_states, Wg, W1, b1, W2, b2):
    Bv, Tv, D = hidden_states.shape
    N = Bv * Tv
    E, H, _ = W1.shape
    cap = math.ceil(_CAP_FACTOR * N / E)
    HC = 1024
    K = H // HC

    flat = hidden_states.reshape(N, D)

    g, pos = pl.pallas_call(
        _router_kernel,
        grid=(1,),
        in_specs=[
            pl.BlockSpec((N, D), lambda i: (0, 0)),
            pl.BlockSpec((E, D), lambda i: (0, 0)),
        ],
        out_specs=[
            pl.BlockSpec((E, N), lambda i: (0, 0)),
            pl.BlockSpec((E, N), lambda i: (0, 0)),
        ],
        out_shape=[
            jax.ShapeDtypeStruct((E, N), jnp.float32),
            jax.ShapeDtypeStruct((E, N), jnp.int32),
        ],
    )(flat, Wg)

    final = pl.pallas_call(
        _expert_kernel,
        grid=(E, K),
        in_specs=[
            pl.BlockSpec((N, D), lambda e, k: (0, 0)),
            pl.BlockSpec((E, N), lambda e, k: (0, 0)),
            pl.BlockSpec((E, N), lambda e, k: (0, 0)),
            pl.BlockSpec((1, HC, D), lambda e, k: (e, k, 0)),
            pl.BlockSpec((E, H), lambda e, k: (0, 0)),
            pl.BlockSpec((1, D, HC), lambda e, k: (e, 0, k)),
            pl.BlockSpec((E, D), lambda e, k: (0, 0)),
        ],
        out_specs=pl.BlockSpec((N, D), lambda e, k: (0, 0)),
        out_shape=jax.ShapeDtypeStruct((N, D), jnp.float32),
        scratch_shapes=[
            pltpu.VMEM((N, D), jnp.bfloat16),
            pltpu.VMEM((cap, N), jnp.bfloat16),
            pltpu.VMEM((cap, D), jnp.bfloat16),
            pltpu.VMEM((cap, D), jnp.float32),
            pltpu.VMEM((cap, 1), jnp.float32),
        ],
    )(flat, g, pos, W1, b1, W2, b2)

    aux_loss = jnp.asarray(0.0, dtype=jnp.float32)
    return (final.reshape(Bv, Tv, D), aux_loss)


# speedup vs baseline: 1.4334x; 1.0276x over previous
import math

import jax
import jax.numpy as jnp
from jax.experimental import pallas as pl
from jax.experimental.pallas import tpu as pltpu

_TOPK = 2
_CAP_FACTOR = 2.0


def _moe_kernel(flat_ref, wg_ref, w1_ref, b1_ref, w2_ref, b2_ref,
                out_ref, g_scr, pos_scr, fbf_scr, p_scr, gath_scr,
                acc_scr, gv_scr):
    e = pl.program_id(0)
    k = pl.program_id(1)
    K = pl.num_programs(1)
    HC = w1_ref.shape[1]
    cap = p_scr.shape[0]
    E, N = g_scr.shape

    @pl.when((e == 0) & (k == 0))
    def _route():
        logits = jax.lax.dot_general(
            wg_ref[...], flat_ref[...],
            dimension_numbers=(((1,), (1,)), ((), ())),
            preferred_element_type=jnp.float32)
        m = jnp.max(logits, axis=0, keepdims=True)
        ex = jnp.exp(logits - m)
        g = ex / jnp.sum(ex, axis=0, keepdims=True)
        ioe = jax.lax.broadcasted_iota(jnp.int32, (E, N), 0)
        v1 = jnp.max(g, axis=0, keepdims=True)
        i1 = jnp.min(jnp.where(g == v1, ioe, E), axis=0, keepdims=True)
        m1 = ioe == i1
        g2 = jnp.where(m1, -jnp.inf, g)
        v2 = jnp.max(g2, axis=0, keepdims=True)
        i2 = jnp.min(jnp.where(g2 == v2, ioe, E), axis=0, keepdims=True)
        mask = (m1 | (ioe == i2)).astype(jnp.int32)
        cums = mask
        sh = 1
        while sh < N:
            shifted = jnp.concatenate(
                [jnp.zeros((E, sh), jnp.int32), cums[:, :N - sh]], axis=1)
            cums = cums + shifted
            sh *= 2
        pos_scr[...] = cums * mask - 1
        g_scr[...] = g
        fbf_scr[...] = flat_ref[...].astype(jnp.bfloat16)
        out_ref[...] = jnp.zeros_like(out_ref)

    @pl.when(k == 0)
    def _dispatch():
        pos_row = pos_scr[pl.ds(e, 1), :]
        ioc = jax.lax.broadcasted_iota(jnp.int32, (cap, N), 0)
        pf = (ioc == pos_row).astype(jnp.float32)
        p_scr[...] = pf.astype(jnp.bfloat16)
        gv_scr[...] = jnp.sum(
            pf * g_scr[pl.ds(e, 1), :], axis=1, keepdims=True)
        gath_scr[...] = jax.lax.dot_general(
            p_scr[...], fbf_scr[...],
            dimension_numbers=(((1,), (0,)), ((), ())),
            preferred_element_type=jnp.float32).astype(jnp.bfloat16)
        acc_scr[...] = jnp.zeros_like(acc_scr)

    hpre = jax.lax.dot_general(
        gath_scr[...], w1_ref[0].astype(jnp.bfloat16),
        dimension_numbers=(((1,), (1,)), ((), ())),
        preferred_element_type=jnp.float32)
    hpre = hpre + b1_ref[pl.ds(e, 1), pl.ds(k * HC, HC)]
    h = 0.5 * hpre * (1.0 + jax.lax.erf(hpre * (1.0 / math.sqrt(2.0))))
    acc_scr[...] += jax.lax.dot_general(
        h.astype(jnp.bfloat16), w2_ref[0].astype(jnp.bfloat16),
        dimension_numbers=(((1,), (1,)), ((), ())),
        preferred_element_type=jnp.float32)

    @pl.when(k == K - 1)
    def _combine():
        outc = (acc_scr[...] + b2_ref[pl.ds(e, 1), :]) * gv_scr[...]
        out_ref[...] += jax.lax.dot_general(
            p_scr[...], outc.astype(jnp.bfloat16),
            dimension_numbers=(((0,), (0,)), ((), ())),
            preferred_element_type=jnp.float32)


def kernel(hidden_states, Wg, W1, b1, W2, b2):
    Bv, Tv, D = hidden_states.shape
    N = Bv * Tv
    E, H, _ = W1.shape
    cap = math.ceil(_CAP_FACTOR * N / E)
    HC = 1024
    K = H // HC

    flat = hidden_states.reshape(N, D)

    final = pl.pallas_call(
        _moe_kernel,
        grid=(E, K),
        in_specs=[
            pl.BlockSpec((N, D), lambda e, k: (0, 0)),
            pl.BlockSpec((E, D), lambda e, k: (0, 0)),
            pl.BlockSpec((1, HC, D), lambda e, k: (e, k, 0)),
            pl.BlockSpec((E, H), lambda e, k: (0, 0)),
            pl.BlockSpec((1, D, HC), lambda e, k: (e, 0, k)),
            pl.BlockSpec((E, D), lambda e, k: (0, 0)),
        ],
        out_specs=pl.BlockSpec((N, D), lambda e, k: (0, 0)),
        out_shape=jax.ShapeDtypeStruct((N, D), jnp.float32),
        scratch_shapes=[
            pltpu.VMEM((E, N), jnp.float32),
            pltpu.VMEM((E, N), jnp.int32),
            pltpu.VMEM((N, D), jnp.bfloat16),
            pltpu.VMEM((cap, N), jnp.bfloat16),
            pltpu.VMEM((cap, D), jnp.bfloat16),
            pltpu.VMEM((cap, D), jnp.float32),
            pltpu.VMEM((cap, 1), jnp.float32),
        ],
    )(flat, Wg, W1, b1, W2, b2)

    aux_loss = jnp.asarray(0.0, dtype=jnp.float32)
    return (final.reshape(Bv, Tv, D), aux_loss)
